# baseline (speedup 1.0000x reference)
import functools

import jax
import jax.numpy as jnp
from jax.experimental import pallas as pl
from jax.experimental.pallas import tpu as pltpu


def _linear_kernel(x_ref, w_ref, b_ref, o_ref):
    o_ref[...] = (
        jnp.dot(x_ref[...], w_ref[...], preferred_element_type=jnp.float32)
        + b_ref[...]
    )


@functools.partial(jax.jit, static_argnames=("blk",))
def _apply_linear(table, wt, b2d, blk):
    rows, dim = table.shape
    grid = (rows // blk,)
    return pl.pallas_call(
        _linear_kernel,
        grid=grid,
        in_specs=[
            pl.BlockSpec((blk, dim), lambda i: (i, 0)),
            pl.BlockSpec((dim, dim), lambda i: (0, 0)),
            pl.BlockSpec((1, dim), lambda i: (0, 0)),
        ],
        out_specs=pl.BlockSpec((blk, dim), lambda i: (i, 0)),
        out_shape=jax.ShapeDtypeStruct((rows, dim), jnp.float32),
        compiler_params=pltpu.CompilerParams(
            dimension_semantics=("parallel",),
        ),
    )(table, wt, b2d)


def kernel(node_ids, edge_idx, node_table, edge_table, W_lin, b_lin):
    wt = W_lin.T
    b2d = b_lin.reshape(1, -1)
    x = _apply_linear(node_table, wt, b2d, blk=10000)
    e = _apply_linear(edge_table, wt, b2d, blk=20000)
    return (x, e)


# speedup vs baseline: 1.0072x; 1.0072x over previous
import functools

import jax
import jax.numpy as jnp
from jax.experimental import pallas as pl
from jax.experimental.pallas import tpu as pltpu

_NODE_BLK = 10000
_EDGE_BLK = 20000


def _linear_kernel(node_ref, edge_ref, w_ref, b_ref, x_ref, e_ref):
    i = pl.program_id(0)

    @pl.when(i == 0)
    def _node():
        x_ref[...] = (
            jnp.dot(node_ref[...], w_ref[...], preferred_element_type=jnp.float32)
            + b_ref[...]
        )

    @pl.when(i > 0)
    def _edge():
        e_ref[...] = (
            jnp.dot(edge_ref[...], w_ref[...], preferred_element_type=jnp.float32)
            + b_ref[...]
        )


@jax.jit
def _fused_linear(node_table, edge_table, wt, b2d):
    n_rows, dim = node_table.shape
    e_rows, _ = edge_table.shape
    n_edge_blocks = e_rows // _EDGE_BLK
    grid = (1 + n_edge_blocks,)
    return pl.pallas_call(
        _linear_kernel,
        grid=grid,
        in_specs=[
            pl.BlockSpec((_NODE_BLK, dim), lambda i: (0, 0)),
            pl.BlockSpec((_EDGE_BLK, dim), lambda i: (jnp.maximum(i - 1, 0), 0)),
            pl.BlockSpec((dim, dim), lambda i: (0, 0)),
            pl.BlockSpec((1, dim), lambda i: (0, 0)),
        ],
        out_specs=[
            pl.BlockSpec((_NODE_BLK, dim), lambda i: (0, 0)),
            pl.BlockSpec((_EDGE_BLK, dim), lambda i: (jnp.maximum(i - 1, 0), 0)),
        ],
        out_shape=[
            jax.ShapeDtypeStruct((n_rows, dim), jnp.float32),
            jax.ShapeDtypeStruct((e_rows, dim), jnp.float32),
        ],
        compiler_params=pltpu.CompilerParams(
            dimension_semantics=("arbitrary",),
        ),
    )(node_table, edge_table, wt, b2d)


def kernel(node_ids, edge_idx, node_table, edge_table, W_lin, b_lin):
    wt = W_lin.T
    b2d = b_lin.reshape(1, -1)
    x, e = _fused_linear(node_table, edge_table, wt, b2d)
    return (x, e)


# speedup vs baseline: 1.0078x; 1.0006x over previous
import functools

import jax
import jax.numpy as jnp
from jax.experimental import pallas as pl
from jax.experimental.pallas import tpu as pltpu

_NODE_BLK = 10000
_EDGE_BLK = 20000


def _linear_kernel(node_ref, edge_ref, w_ref, b_ref, x_ref, e_ref):
    i = pl.program_id(0)

    @pl.when(i == 0)
    def _node():
        x_ref[...] = (
            jnp.dot(node_ref[...], w_ref[...], preferred_element_type=jnp.float32)
            + b_ref[...]
        )

    @pl.when(i > 0)
    def _edge():
        e_ref[...] = (
            jnp.dot(edge_ref[...], w_ref[...], preferred_element_type=jnp.float32)
            + b_ref[...]
        )


@jax.jit
def _fused_linear(node_table, edge_table, wt, b2d):
    n_rows, dim = node_table.shape
    e_rows, _ = edge_table.shape
    n_edge_blocks = e_rows // _EDGE_BLK
    grid = (1 + n_edge_blocks,)
    return pl.pallas_call(
        _linear_kernel,
        grid=grid,
        in_specs=[
            pl.BlockSpec((_NODE_BLK, dim), lambda i: (0, 0)),
            pl.BlockSpec((_EDGE_BLK, dim), lambda i: (jnp.maximum(i - 1, 0), 0)),
            pl.BlockSpec((dim, dim), lambda i: (0, 0)),
            pl.BlockSpec((1, dim), lambda i: (0, 0)),
        ],
        out_specs=[
            pl.BlockSpec((_NODE_BLK, dim), lambda i: (0, 0)),
            pl.BlockSpec((_EDGE_BLK, dim), lambda i: (jnp.maximum(i - 1, 0), 0)),
        ],
        out_shape=[
            jax.ShapeDtypeStruct((n_rows, dim), jnp.float32),
            jax.ShapeDtypeStruct((e_rows, dim), jnp.float32),
        ],
        compiler_params=pltpu.CompilerParams(
            dimension_semantics=("parallel",),
        ),
    )(node_table, edge_table, wt, b2d)


def kernel(node_ids, edge_idx, node_table, edge_table, W_lin, b_lin):
    wt = W_lin.T
    b2d = b_lin.reshape(1, -1)
    x, e = _fused_linear(node_table, edge_table, wt, b2d)
    return (x, e)
